# baseline (speedup 1.0000x reference)
import jax
import jax.numpy as jnp
from jax import lax
from jax.experimental import pallas as pl
from jax.experimental.pallas import tpu as pltpu

EMBED_DIM = 768
FF_DIM = 3072
NUM_EXPERTS = 8
TOP_K = 2
CAPACITY_FACTOR = 2.0
LOAD_BALANCE_WEIGHT = 0.01
ROUTER_Z_WEIGHT = 0.001

BT = 512
BF = 768
NF = FF_DIM // BF


def _ffn_body(offs_ref, nbe_ref, x_ref, r0_ref, r1_ref, w0_ref, w1_ref,
              wg_ref, wu_ref, wo_ref, out_ref, xg_ref, yacc_ref):
    e = pl.program_id(0)
    b = pl.program_id(1)
    j = pl.program_id(2)
    active = b < nbe_ref[e]
    base = offs_ref[e] + b * BT
    T = x_ref.shape[0]

    @pl.when((e == 0) & (b == 0) & (j == 0))
    def _init():
        out_ref[:, :] = jnp.zeros_like(out_ref)

    @pl.when(active)
    def _compute():
        @pl.when(j == 0)
        def _gather():
            rows = lax.broadcasted_iota(jnp.int32, (BT, T), 0) + base
            m = ((rows == r0_ref[0][None, :])
                 | (rows == r1_ref[0][None, :])).astype(jnp.bfloat16)
            xg_ref[:, :] = jnp.dot(
                m, x_ref[:, :],
                preferred_element_type=jnp.float32).astype(jnp.bfloat16)

        xb = xg_ref[:, :]
        wg = wg_ref[0]
        wu = wu_ref[0]
        wo = wo_ref[0]
        dn = (((1,), (1,)), ((), ()))
        g = lax.dot_general(xb, wg, dn, preferred_element_type=jnp.float32)
        u = lax.dot_general(xb, wu, dn, preferred_element_type=jnp.float32)
        h = (g * jax.nn.sigmoid(g)) * u
        yb = lax.dot_general(h.astype(jnp.bfloat16), wo, dn,
                             preferred_element_type=jnp.float32)

        @pl.when(j == 0)
        def _():
            yacc_ref[:, :] = yb

        @pl.when(j > 0)
        def _():
            yacc_ref[:, :] = yacc_ref[:, :] + yb

        @pl.when(j == NF - 1)
        def _scatter():
            rows = lax.broadcasted_iota(jnp.int32, (BT, T), 0) + base
            mw = ((rows == r0_ref[0][None, :]).astype(jnp.float32)
                  * w0_ref[0][None, :]
                  + (rows == r1_ref[0][None, :]).astype(jnp.float32)
                  * w1_ref[0][None, :])
            dns = (((0,), (0,)), ((), ()))
            out_ref[:, :] = out_ref[:, :] + lax.dot_general(
                mw, yacc_ref[:, :], dns, preferred_element_type=jnp.float32)


def _moe_ffn(xf, row0, row1, w0k, w1k, offs, nbe, wi_gate, wi_up, wo, nbe_max):
    T, D = xf.shape

    def jeff(b, j, nbe, e):
        return jnp.where(b < nbe[e], j, NF - 1)

    grid_spec = pltpu.PrefetchScalarGridSpec(
        num_scalar_prefetch=2,
        grid=(NUM_EXPERTS, nbe_max, NF),
        in_specs=[
            pl.BlockSpec((T, D), lambda e, b, j, offs, nbe: (0, 0)),
            pl.BlockSpec((1, T), lambda e, b, j, offs, nbe: (0, 0)),
            pl.BlockSpec((1, T), lambda e, b, j, offs, nbe: (0, 0)),
            pl.BlockSpec((1, T), lambda e, b, j, offs, nbe: (0, 0)),
            pl.BlockSpec((1, T), lambda e, b, j, offs, nbe: (0, 0)),
            pl.BlockSpec((1, BF, D),
                         lambda e, b, j, offs, nbe: (e, jeff(b, j, nbe, e), 0)),
            pl.BlockSpec((1, BF, D),
                         lambda e, b, j, offs, nbe: (e, jeff(b, j, nbe, e), 0)),
            pl.BlockSpec((1, D, BF),
                         lambda e, b, j, offs, nbe: (e, 0, jeff(b, j, nbe, e))),
        ],
        out_specs=pl.BlockSpec((T, D), lambda e, b, j, offs, nbe: (0, 0)),
        scratch_shapes=[
            pltpu.VMEM((BT, D), jnp.bfloat16),
            pltpu.VMEM((BT, D), jnp.float32),
        ],
    )
    return pl.pallas_call(
        _ffn_body,
        grid_spec=grid_spec,
        out_shape=jax.ShapeDtypeStruct((T, D), jnp.float32),
    )(offs, nbe, xf.astype(jnp.bfloat16), row0.reshape(1, T),
      row1.reshape(1, T), w0k.reshape(1, T), w1k.reshape(1, T),
      wi_gate.astype(jnp.bfloat16), wi_up.astype(jnp.bfloat16),
      wo.astype(jnp.bfloat16))


def kernel(x, gate_w, wi_gate, wi_up, wo):
    B, S, D = x.shape
    T = B * S
    E = NUM_EXPERTS
    cap = max(int(T * TOP_K / E * CAPACITY_FACTOR), TOP_K)
    xf = x.reshape(T, D)

    logits = xf @ gate_w.T
    probs = jax.nn.softmax(logits, axis=-1)
    i0 = jnp.argmax(probs, axis=-1)
    p0 = jnp.max(probs, axis=-1)
    e_ids = jnp.arange(E, dtype=jnp.int32)
    masked = jnp.where(i0[:, None] == e_ids[None, :], -jnp.inf, probs)
    i1 = jnp.argmax(masked, axis=-1)
    p1 = jnp.max(masked, axis=-1)
    s = p0 + p1
    w0 = p0 / s
    w1 = p1 / s

    oh0 = (i0[:, None] == e_ids[None, :]).astype(jnp.int32)
    oh1 = (i1[:, None] == e_ids[None, :]).astype(jnp.int32)
    cum0 = jnp.cumsum(oh0, axis=0)
    cum1 = jnp.cumsum(oh1, axis=0)
    rank0 = jnp.sum(cum0 * oh0, axis=1)
    rank1 = jnp.sum(cum1 * oh1, axis=1)
    kept0 = rank0 <= cap
    kept1 = rank1 <= cap
    cnt0 = jnp.sum((cum0 <= cap) * oh0, axis=0)
    cnt1 = jnp.sum((cum1 <= cap) * oh1, axis=0)
    size = cnt0 + cnt1
    padded = ((size + BT - 1) // BT) * BT
    offs = jnp.concatenate([jnp.zeros((1,), jnp.int32),
                            jnp.cumsum(padded)[:-1].astype(jnp.int32)])
    nbe = (padded // BT).astype(jnp.int32)
    nbe_max = (2 * cap + BT - 1) // BT

    sentinel = T * TOP_K + E * BT
    row0 = jnp.where(kept0, offs[i0] + rank0 - 1, sentinel).astype(jnp.int32)
    row1 = jnp.where(kept1, offs[i1] + cnt0[i1] + rank1 - 1,
                     sentinel).astype(jnp.int32)
    w0k = jnp.where(kept0, w0, 0.0)
    w1k = jnp.where(kept1, w1, 0.0)

    out = _moe_ffn(xf, row0, row1, w0k, w1k, offs, nbe,
                   wi_gate, wi_up, wo, nbe_max)
    output = out.reshape(B, S, D)

    f = (oh0 + oh1).sum(axis=0).astype(jnp.float32) / (T * TOP_K)
    P = probs.mean(axis=0)
    load_balance_loss = E * jnp.sum(f * P)
    lse = jax.scipy.special.logsumexp(logits, axis=-1)
    z_loss = jnp.mean(jnp.square(lse))
    aux_loss = (LOAD_BALANCE_WEIGHT * load_balance_loss
                + ROUTER_Z_WEIGHT * z_loss)
    return (output, aux_loss)


# speedup vs baseline: 1.4276x; 1.4276x over previous
import jax
import jax.numpy as jnp
from jax import lax
from jax.experimental import pallas as pl
from jax.experimental.pallas import tpu as pltpu

EMBED_DIM = 768
FF_DIM = 3072
NUM_EXPERTS = 8
TOP_K = 2
CAPACITY_FACTOR = 2.0
LOAD_BALANCE_WEIGHT = 0.01
ROUTER_Z_WEIGHT = 0.001

BT = 512
BF = 768
NF = FF_DIM // BF


def _ffn_body(offs_ref, nbe_ref, x_ref, r0_ref, r1_ref, w0_ref, w1_ref,
              wg_ref, wu_ref, wo_ref, out_ref, xg_ref, yacc_ref):
    e = pl.program_id(0)
    b = pl.program_id(1)
    j = pl.program_id(2)
    active = b < nbe_ref[e]
    base = offs_ref[e] + b * BT
    T = x_ref.shape[0]

    @pl.when((e == 0) & (b == 0) & (j == 0))
    def _init():
        out_ref[:, :] = jnp.zeros_like(out_ref)

    @pl.when(active)
    def _compute():
        @pl.when(j == 0)
        def _gather():
            rows = lax.broadcasted_iota(jnp.int32, (BT, T), 0) + base
            m = ((rows == r0_ref[0][None, :]).astype(jnp.float32)
                 + (rows == r1_ref[0][None, :]).astype(jnp.float32))
            xg_ref[:, :] = jnp.dot(m, x_ref[:, :],
                                   preferred_element_type=jnp.float32)

        xb = xg_ref[:, :]
        wg = wg_ref[0]
        wu = wu_ref[0]
        wo = wo_ref[0]
        dn = (((1,), (1,)), ((), ()))
        g = lax.dot_general(xb, wg, dn, preferred_element_type=jnp.float32)
        u = lax.dot_general(xb, wu, dn, preferred_element_type=jnp.float32)
        h = (g * jax.nn.sigmoid(g)) * u
        yb = lax.dot_general(h, wo, dn, preferred_element_type=jnp.float32)

        @pl.when(j == 0)
        def _():
            yacc_ref[:, :] = yb

        @pl.when(j > 0)
        def _():
            yacc_ref[:, :] = yacc_ref[:, :] + yb

        @pl.when(j == NF - 1)
        def _scatter():
            rows = lax.broadcasted_iota(jnp.int32, (BT, T), 0) + base
            mw = ((rows == r0_ref[0][None, :]).astype(jnp.float32)
                  * w0_ref[0][None, :]
                  + (rows == r1_ref[0][None, :]).astype(jnp.float32)
                  * w1_ref[0][None, :])
            dns = (((0,), (0,)), ((), ()))
            out_ref[:, :] = out_ref[:, :] + lax.dot_general(
                mw, yacc_ref[:, :], dns, preferred_element_type=jnp.float32)


def _moe_ffn(xf, row0, row1, w0k, w1k, offs, nbe, wi_gate, wi_up, wo, nbe_max):
    T, D = xf.shape

    def jeff(b, j, nbe, e):
        return jnp.where(b < nbe[e], j, NF - 1)

    grid_spec = pltpu.PrefetchScalarGridSpec(
        num_scalar_prefetch=2,
        grid=(NUM_EXPERTS, nbe_max, NF),
        in_specs=[
            pl.BlockSpec((T, D), lambda e, b, j, offs, nbe: (0, 0)),
            pl.BlockSpec((1, T), lambda e, b, j, offs, nbe: (0, 0)),
            pl.BlockSpec((1, T), lambda e, b, j, offs, nbe: (0, 0)),
            pl.BlockSpec((1, T), lambda e, b, j, offs, nbe: (0, 0)),
            pl.BlockSpec((1, T), lambda e, b, j, offs, nbe: (0, 0)),
            pl.BlockSpec((1, BF, D),
                         lambda e, b, j, offs, nbe: (e, jeff(b, j, nbe, e), 0)),
            pl.BlockSpec((1, BF, D),
                         lambda e, b, j, offs, nbe: (e, jeff(b, j, nbe, e), 0)),
            pl.BlockSpec((1, D, BF),
                         lambda e, b, j, offs, nbe: (e, 0, jeff(b, j, nbe, e))),
        ],
        out_specs=pl.BlockSpec((T, D), lambda e, b, j, offs, nbe: (0, 0)),
        scratch_shapes=[
            pltpu.VMEM((BT, D), jnp.float32),
            pltpu.VMEM((BT, D), jnp.float32),
        ],
    )
    return pl.pallas_call(
        _ffn_body,
        grid_spec=grid_spec,
        out_shape=jax.ShapeDtypeStruct((T, D), jnp.float32),
    )(offs, nbe, xf, row0.reshape(1, T), row1.reshape(1, T),
      w0k.reshape(1, T), w1k.reshape(1, T), wi_gate, wi_up, wo)


LANES = 128


def _router_body(x_ref, gw_ref, r0_ref, r1_ref, w0_ref, w1_ref,
                 offs_ref, nbe_ref, aux_ref):
    T = x_ref.shape[0]
    E = NUM_EXPERTS
    cap = max(int(T * TOP_K / E * CAPACITY_FACTOR), TOP_K)
    sentinel = T * TOP_K + E * BT

    xv = x_ref[:, :]
    gw = gw_ref[:, :]
    dn = (((1,), (1,)), ((), ()))
    logits = lax.dot_general(xv, gw, dn,
                             preferred_element_type=jnp.float32)
    lane = lax.broadcasted_iota(jnp.int32, (T, LANES), 1)
    valid = lane < E
    lm = jnp.where(valid, logits, -jnp.inf)
    ml = jnp.max(lm, axis=1, keepdims=True)
    ex = jnp.where(valid, jnp.exp(lm - ml), 0.0)
    se = jnp.sum(ex, axis=1, keepdims=True)
    probs = ex / se
    lse = ml[:, 0] + jnp.log(se[:, 0])

    p0 = jnp.max(probs, axis=1, keepdims=True)
    i0 = jnp.min(jnp.where(probs == p0, lane, LANES), axis=1, keepdims=True)
    masked = jnp.where(lane == i0, -jnp.inf, jnp.where(valid, probs, -jnp.inf))
    p1 = jnp.max(masked, axis=1, keepdims=True)
    i1 = jnp.min(jnp.where(masked == p1, lane, LANES), axis=1, keepdims=True)
    s = p0 + p1
    w0 = (p0 / s)[:, 0]
    w1 = (p1 / s)[:, 0]

    oh0 = (lane == i0).astype(jnp.float32)
    oh1 = (lane == i1).astype(jnp.float32)
    tri = (lax.broadcasted_iota(jnp.int32, (T, T), 0)
           >= lax.broadcasted_iota(jnp.int32, (T, T), 1)).astype(jnp.float32)
    cum0 = jnp.dot(tri, oh0, preferred_element_type=jnp.float32)
    cum1 = jnp.dot(tri, oh1, preferred_element_type=jnp.float32)
    rank0 = jnp.sum(cum0 * oh0, axis=1).astype(jnp.int32)
    rank1 = jnp.sum(cum1 * oh1, axis=1).astype(jnp.int32)
    kept0 = rank0 <= cap
    kept1 = rank1 <= cap
    cnt0 = jnp.sum(jnp.where(cum0 <= cap, oh0, 0.0), axis=0)
    cnt1 = jnp.sum(jnp.where(cum1 <= cap, oh1, 0.0), axis=0)
    size = (cnt0 + cnt1).astype(jnp.int32)
    padded = ((size + BT - 1) // BT) * BT
    ltri = (lax.broadcasted_iota(jnp.int32, (LANES, LANES), 0)
            < lax.broadcasted_iota(jnp.int32, (LANES, LANES), 1)
            ).astype(jnp.float32)
    offs = jnp.dot(padded.astype(jnp.float32).reshape(1, LANES), ltri,
                   preferred_element_type=jnp.float32)[0].astype(jnp.int32)
    nbe = padded // BT

    off_i0 = jnp.sum(oh0 * offs.astype(jnp.float32)[None, :], axis=1)
    off_i1 = jnp.sum(oh1 * offs.astype(jnp.float32)[None, :], axis=1)
    cnt0_i1 = jnp.sum(oh1 * cnt0[None, :], axis=1)
    row0 = jnp.where(kept0,
                     off_i0.astype(jnp.int32) + rank0 - 1, sentinel)
    row1 = jnp.where(kept1,
                     off_i1.astype(jnp.int32) + cnt0_i1.astype(jnp.int32)
                     + rank1 - 1, sentinel)

    r0_ref[:, :] = row0[None, :]
    r1_ref[:, :] = row1[None, :]
    w0_ref[:, :] = jnp.where(kept0, w0, 0.0)[None, :]
    w1_ref[:, :] = jnp.where(kept1, w1, 0.0)[None, :]
    offs_ref[:, :] = offs[None, :]
    nbe_ref[:, :] = nbe[None, :]

    f = jnp.sum(oh0 + oh1, axis=0) / (T * TOP_K)
    P = jnp.mean(probs, axis=0)
    lb = E * jnp.sum(f * P)
    z = jnp.mean(jnp.square(lse))
    aux = LOAD_BALANCE_WEIGHT * lb + ROUTER_Z_WEIGHT * z
    aux_ref[:, :] = jnp.full((1, LANES), aux, jnp.float32)


def _router(xf, gate_w):
    T, D = xf.shape
    gwp = jnp.zeros((LANES, D), jnp.float32).at[:NUM_EXPERTS].set(gate_w)
    outs = pl.pallas_call(
        _router_body,
        grid=(1,),
        in_specs=[
            pl.BlockSpec((T, D), lambda i: (0, 0)),
            pl.BlockSpec((LANES, D), lambda i: (0, 0)),
        ],
        out_specs=[
            pl.BlockSpec((1, T), lambda i: (0, 0)),
            pl.BlockSpec((1, T), lambda i: (0, 0)),
            pl.BlockSpec((1, T), lambda i: (0, 0)),
            pl.BlockSpec((1, T), lambda i: (0, 0)),
            pl.BlockSpec((1, LANES), lambda i: (0, 0)),
            pl.BlockSpec((1, LANES), lambda i: (0, 0)),
            pl.BlockSpec((1, LANES), lambda i: (0, 0)),
        ],
        out_shape=[
            jax.ShapeDtypeStruct((1, T), jnp.int32),
            jax.ShapeDtypeStruct((1, T), jnp.int32),
            jax.ShapeDtypeStruct((1, T), jnp.float32),
            jax.ShapeDtypeStruct((1, T), jnp.float32),
            jax.ShapeDtypeStruct((1, LANES), jnp.int32),
            jax.ShapeDtypeStruct((1, LANES), jnp.int32),
            jax.ShapeDtypeStruct((1, LANES), jnp.float32),
        ],
    )(xf, gwp)
    return outs


def kernel(x, gate_w, wi_gate, wi_up, wo):
    B, S, D = x.shape
    T = B * S
    E = NUM_EXPERTS
    cap = max(int(T * TOP_K / E * CAPACITY_FACTOR), TOP_K)
    xf = x.reshape(T, D)

    row0, row1, w0k, w1k, offs_l, nbe_l, auxv = _router(xf, gate_w)
    offs = offs_l[0, :E]
    nbe = nbe_l[0, :E]
    nbe_max = (2 * cap + BT - 1) // BT

    out = _moe_ffn(xf, row0[0], row1[0], w0k[0], w1k[0], offs, nbe,
                   wi_gate, wi_up, wo, nbe_max)
    output = out.reshape(B, S, D)
    return (output, auxv[0, 0])


# speedup vs baseline: 1.4358x; 1.0058x over previous
import jax
import jax.numpy as jnp
from jax import lax
from jax.experimental import pallas as pl
from jax.experimental.pallas import tpu as pltpu

EMBED_DIM = 768
FF_DIM = 3072
NUM_EXPERTS = 8
TOP_K = 2
CAPACITY_FACTOR = 2.0
LOAD_BALANCE_WEIGHT = 0.01
ROUTER_Z_WEIGHT = 0.001

BT = 512
BF = 768
NF = FF_DIM // BF


def _ffn_body(offs_ref, nbe_ref, x_ref, r0_ref, r1_ref, w0_ref, w1_ref,
              wg_ref, wu_ref, wo_ref, out_ref, xg_ref, yacc_ref,
              m_ref, wrow_ref):
    e = pl.program_id(0)
    b = pl.program_id(1)
    j = pl.program_id(2)
    active = b < nbe_ref[e]
    base = offs_ref[e] + b * BT
    T = x_ref.shape[0]

    @pl.when((e == 0) & (b == 0) & (j == 0))
    def _init():
        out_ref[:, :] = jnp.zeros_like(out_ref)

    @pl.when(active)
    def _compute():
        @pl.when(j == 0)
        def _gather():
            rows = lax.broadcasted_iota(jnp.int32, (BT, T), 0) + base
            m0 = (rows == r0_ref[0][None, :]).astype(jnp.float32)
            m1 = (rows == r1_ref[0][None, :]).astype(jnp.float32)
            m_ref[:, :] = m0 + m1
            wrow_ref[:, :] = (
                jnp.sum(m0 * w0_ref[0][None, :], axis=1, keepdims=True)
                + jnp.sum(m1 * w1_ref[0][None, :], axis=1, keepdims=True))
            xg_ref[:, :] = jnp.dot(m_ref[:, :], x_ref[:, :],
                                   preferred_element_type=jnp.float32)

        xb = xg_ref[:, :]
        wg = wg_ref[0]
        wu = wu_ref[0]
        wo = wo_ref[0]
        dn = (((1,), (1,)), ((), ()))
        g = lax.dot_general(xb, wg, dn, preferred_element_type=jnp.float32)
        u = lax.dot_general(xb, wu, dn, preferred_element_type=jnp.float32)
        h = (g * jax.nn.sigmoid(g)) * u
        yb = lax.dot_general(h, wo, dn, preferred_element_type=jnp.float32)

        @pl.when(j == 0)
        def _():
            yacc_ref[:, :] = yb

        @pl.when(j > 0)
        def _():
            yacc_ref[:, :] = yacc_ref[:, :] + yb

        @pl.when(j == NF - 1)
        def _scatter():
            dns = (((0,), (0,)), ((), ()))
            out_ref[:, :] = out_ref[:, :] + lax.dot_general(
                m_ref[:, :], yacc_ref[:, :] * wrow_ref[:, :], dns,
                preferred_element_type=jnp.float32)


def _moe_ffn(xf, row0, row1, w0k, w1k, offs, nbe, wi_gate, wi_up, wo, nbe_max):
    T, D = xf.shape

    def jeff(b, j, nbe, e):
        return jnp.where(b < nbe[e], j, NF - 1)

    grid_spec = pltpu.PrefetchScalarGridSpec(
        num_scalar_prefetch=2,
        grid=(NUM_EXPERTS, nbe_max, NF),
        in_specs=[
            pl.BlockSpec((T, D), lambda e, b, j, offs, nbe: (0, 0)),
            pl.BlockSpec((1, T), lambda e, b, j, offs, nbe: (0, 0)),
            pl.BlockSpec((1, T), lambda e, b, j, offs, nbe: (0, 0)),
            pl.BlockSpec((1, T), lambda e, b, j, offs, nbe: (0, 0)),
            pl.BlockSpec((1, T), lambda e, b, j, offs, nbe: (0, 0)),
            pl.BlockSpec((1, BF, D),
                         lambda e, b, j, offs, nbe: (e, jeff(b, j, nbe, e), 0)),
            pl.BlockSpec((1, BF, D),
                         lambda e, b, j, offs, nbe: (e, jeff(b, j, nbe, e), 0)),
            pl.BlockSpec((1, D, BF),
                         lambda e, b, j, offs, nbe: (e, 0, jeff(b, j, nbe, e))),
        ],
        out_specs=pl.BlockSpec((T, D), lambda e, b, j, offs, nbe: (0, 0)),
        scratch_shapes=[
            pltpu.VMEM((BT, D), jnp.float32),
            pltpu.VMEM((BT, D), jnp.float32),
            pltpu.VMEM((BT, T), jnp.float32),
            pltpu.VMEM((BT, 1), jnp.float32),
        ],
    )
    return pl.pallas_call(
        _ffn_body,
        grid_spec=grid_spec,
        out_shape=jax.ShapeDtypeStruct((T, D), jnp.float32),
    )(offs, nbe, xf, row0.reshape(1, T), row1.reshape(1, T),
      w0k.reshape(1, T), w1k.reshape(1, T), wi_gate, wi_up, wo)


LANES = 128


def _router_body(x_ref, gw_ref, r0_ref, r1_ref, w0_ref, w1_ref,
                 offs_ref, nbe_ref, aux_ref):
    T = x_ref.shape[0]
    E = NUM_EXPERTS
    cap = max(int(T * TOP_K / E * CAPACITY_FACTOR), TOP_K)
    sentinel = T * TOP_K + E * BT

    xv = x_ref[:, :]
    gw = gw_ref[:, :]
    dn = (((1,), (1,)), ((), ()))
    logits = lax.dot_general(xv, gw, dn,
                             preferred_element_type=jnp.float32)
    lane = lax.broadcasted_iota(jnp.int32, (T, LANES), 1)
    valid = lane < E
    lm = jnp.where(valid, logits, -jnp.inf)
    ml = jnp.max(lm, axis=1, keepdims=True)
    ex = jnp.where(valid, jnp.exp(lm - ml), 0.0)
    se = jnp.sum(ex, axis=1, keepdims=True)
    probs = ex / se
    lse = ml[:, 0] + jnp.log(se[:, 0])

    p0 = jnp.max(probs, axis=1, keepdims=True)
    i0 = jnp.min(jnp.where(probs == p0, lane, LANES), axis=1, keepdims=True)
    masked = jnp.where(lane == i0, -jnp.inf, jnp.where(valid, probs, -jnp.inf))
    p1 = jnp.max(masked, axis=1, keepdims=True)
    i1 = jnp.min(jnp.where(masked == p1, lane, LANES), axis=1, keepdims=True)
    s = p0 + p1
    w0 = (p0 / s)[:, 0]
    w1 = (p1 / s)[:, 0]

    oh0 = (lane == i0).astype(jnp.float32)
    oh1 = (lane == i1).astype(jnp.float32)
    tri = (lax.broadcasted_iota(jnp.int32, (T, T), 0)
           >= lax.broadcasted_iota(jnp.int32, (T, T), 1)).astype(jnp.float32)
    cum0 = jnp.dot(tri, oh0, preferred_element_type=jnp.float32)
    cum1 = jnp.dot(tri, oh1, preferred_element_type=jnp.float32)
    rank0 = jnp.sum(cum0 * oh0, axis=1).astype(jnp.int32)
    rank1 = jnp.sum(cum1 * oh1, axis=1).astype(jnp.int32)
    kept0 = rank0 <= cap
    kept1 = rank1 <= cap
    cnt0 = jnp.sum(jnp.where(cum0 <= cap, oh0, 0.0), axis=0)
    cnt1 = jnp.sum(jnp.where(cum1 <= cap, oh1, 0.0), axis=0)
    size = (cnt0 + cnt1).astype(jnp.int32)
    padded = ((size + BT - 1) // BT) * BT
    ltri = (lax.broadcasted_iota(jnp.int32, (LANES, LANES), 0)
            < lax.broadcasted_iota(jnp.int32, (LANES, LANES), 1)
            ).astype(jnp.float32)
    offs = jnp.dot(padded.astype(jnp.float32).reshape(1, LANES), ltri,
                   preferred_element_type=jnp.float32)[0].astype(jnp.int32)
    nbe = padded // BT

    off_i0 = jnp.sum(oh0 * offs.astype(jnp.float32)[None, :], axis=1)
    off_i1 = jnp.sum(oh1 * offs.astype(jnp.float32)[None, :], axis=1)
    cnt0_i1 = jnp.sum(oh1 * cnt0[None, :], axis=1)
    row0 = jnp.where(kept0,
                     off_i0.astype(jnp.int32) + rank0 - 1, sentinel)
    row1 = jnp.where(kept1,
                     off_i1.astype(jnp.int32) + cnt0_i1.astype(jnp.int32)
                     + rank1 - 1, sentinel)

    r0_ref[:, :] = row0[None, :]
    r1_ref[:, :] = row1[None, :]
    w0_ref[:, :] = jnp.where(kept0, w0, 0.0)[None, :]
    w1_ref[:, :] = jnp.where(kept1, w1, 0.0)[None, :]
    offs_ref[:, :] = offs[None, :]
    nbe_ref[:, :] = nbe[None, :]

    f = jnp.sum(oh0 + oh1, axis=0) / (T * TOP_K)
    P = jnp.mean(probs, axis=0)
    lb = E * jnp.sum(f * P)
    z = jnp.mean(jnp.square(lse))
    aux = LOAD_BALANCE_WEIGHT * lb + ROUTER_Z_WEIGHT * z
    aux_ref[:, :] = jnp.full((1, LANES), aux, jnp.float32)


def _router(xf, gate_w):
    T, D = xf.shape
    gwp = jnp.zeros((LANES, D), jnp.float32).at[:NUM_EXPERTS].set(gate_w)
    outs = pl.pallas_call(
        _router_body,
        grid=(1,),
        in_specs=[
            pl.BlockSpec((T, D), lambda i: (0, 0)),
            pl.BlockSpec((LANES, D), lambda i: (0, 0)),
        ],
        out_specs=[
            pl.BlockSpec((1, T), lambda i: (0, 0)),
            pl.BlockSpec((1, T), lambda i: (0, 0)),
            pl.BlockSpec((1, T), lambda i: (0, 0)),
            pl.BlockSpec((1, T), lambda i: (0, 0)),
            pl.BlockSpec((1, LANES), lambda i: (0, 0)),
            pl.BlockSpec((1, LANES), lambda i: (0, 0)),
            pl.BlockSpec((1, LANES), lambda i: (0, 0)),
        ],
        out_shape=[
            jax.ShapeDtypeStruct((1, T), jnp.int32),
            jax.ShapeDtypeStruct((1, T), jnp.int32),
            jax.ShapeDtypeStruct((1, T), jnp.float32),
            jax.ShapeDtypeStruct((1, T), jnp.float32),
            jax.ShapeDtypeStruct((1, LANES), jnp.int32),
            jax.ShapeDtypeStruct((1, LANES), jnp.int32),
            jax.ShapeDtypeStruct((1, LANES), jnp.float32),
        ],
    )(xf, gwp)
    return outs


def kernel(x, gate_w, wi_gate, wi_up, wo):
    B, S, D = x.shape
    T = B * S
    E = NUM_EXPERTS
    cap = max(int(T * TOP_K / E * CAPACITY_FACTOR), TOP_K)
    xf = x.reshape(T, D)

    row0, row1, w0k, w1k, offs_l, nbe_l, auxv = _router(xf, gate_w)
    offs = offs_l[0, :E]
    nbe = nbe_l[0, :E]
    nbe_max = (2 * cap + BT - 1) // BT

    out = _moe_ffn(xf, row0[0], row1[0], w0k[0], w1k[0], offs, nbe,
                   wi_gate, wi_up, wo, nbe_max)
    output = out.reshape(B, S, D)
    return (output, auxv[0, 0])
